# baseline (speedup 1.0000x reference)
import functools

import jax
import jax.numpy as jnp
from jax import lax
from jax.experimental import pallas as pl
from jax.experimental.pallas import tpu as pltpu
from jax.experimental.pallas import tpu_sc as plsc

B = 128
R = 2048
C = 128
TOPK_K = 32
BB = 4
L = 16
NV = B // L


def _mse_reduce_body(o_ref, l_ref, out_ref):
    d = o_ref[...] - l_ref[...]
    s = jnp.sum(jnp.sum(d * d, axis=2), axis=1)
    out_ref[0, 0, :] = s * (1.0 / (R * C))


def _losses(output, label):
    grid = (B // BB,)
    out = pl.pallas_call(
        _mse_reduce_body,
        grid=grid,
        in_specs=[
            pl.BlockSpec((BB, R, C), lambda i: (i, 0, 0)),
            pl.BlockSpec((BB, R, C), lambda i: (i, 0, 0)),
        ],
        out_specs=pl.BlockSpec((1, 1, BB), lambda i: (i, 0, 0)),
        out_shape=jax.ShapeDtypeStruct((B // BB, 1, BB), jnp.float32),
    )(output, label)
    return out.reshape(B)


def _topk_sc_body(losses_hbm, out_hbm, vals_v, out_v):
    cid = lax.axis_index("c")
    sid = lax.axis_index("s")

    @pl.when(jnp.logical_and(cid == 0, sid == 0))
    def _():
        pltpu.sync_copy(losses_hbm, vals_v)
        lane = lax.iota(jnp.int32, L)
        v = [vals_v[pl.ds(j * L, L)] for j in range(NV)]
        idx = [lane + j * L for j in range(NV)]
        big = jnp.int32(2 ** 30)
        outs = [jnp.zeros((L,), jnp.float32) for _ in range(TOPK_K // L)]
        for r in range(TOPK_K):
            t = v[0]
            for j in range(1, NV):
                t = jnp.maximum(t, v[j])
            m = jnp.max(t)
            c = jnp.where(v[0] == m, idx[0], big)
            for j in range(1, NV):
                c = jnp.minimum(c, jnp.where(v[j] == m, idx[j], big))
            mi = jnp.min(c)
            for j in range(NV):
                v[j] = jnp.where(idx[j] == mi, jnp.float32(-1.0), v[j])
            q, p = divmod(r, L)
            outs[q] = jnp.where(lane == p, m, outs[q])
        for q in range(TOPK_K // L):
            out_v[pl.ds(q * L, L)] = outs[q]
        pltpu.sync_copy(out_v, out_hbm)


@functools.partial(
    pl.kernel,
    out_type=jax.ShapeDtypeStruct((TOPK_K,), jnp.float32),
    mesh=plsc.VectorSubcoreMesh(core_axis_name="c", subcore_axis_name="s"),
    compiler_params=pltpu.CompilerParams(needs_layout_passes=False),
    scratch_types=[
        pltpu.VMEM((B,), jnp.float32),
        pltpu.VMEM((TOPK_K,), jnp.float32),
    ],
)
def _topk_sc(losses_hbm, out_hbm, vals_v, out_v):
    _topk_sc_body(losses_hbm, out_hbm, vals_v, out_v)


def kernel(output, label):
    return _topk_sc(_losses(output, label))


# speedup vs baseline: 1.0064x; 1.0064x over previous
import functools

import jax
import jax.numpy as jnp
from jax import lax
from jax.experimental import pallas as pl
from jax.experimental.pallas import tpu as pltpu
from jax.experimental.pallas import tpu_sc as plsc

B = 128
R = 2048
C = 128
TOPK_K = 32
BB = 4
L = 16
NV = B // L


def _mse_reduce_body(o_ref, l_ref, out_ref):
    d = o_ref[...] - l_ref[...]
    s = jnp.sum(jnp.sum(d * d, axis=1), axis=1)
    out_ref[0, 0, :] = s * (1.0 / (R * C))


def _losses(output, label):
    grid = (B // BB,)
    out = pl.pallas_call(
        _mse_reduce_body,
        grid=grid,
        in_specs=[
            pl.BlockSpec((BB, R, C), lambda i: (i, 0, 0)),
            pl.BlockSpec((BB, R, C), lambda i: (i, 0, 0)),
        ],
        out_specs=pl.BlockSpec((1, 1, BB), lambda i: (i, 0, 0)),
        out_shape=jax.ShapeDtypeStruct((B // BB, 1, BB), jnp.float32),
    )(output, label)
    return out.reshape(B)


def _topk_sc_body(losses_hbm, out_hbm, vals_v, out_v):
    cid = lax.axis_index("c")
    sid = lax.axis_index("s")

    @pl.when(jnp.logical_and(cid == 0, sid == 0))
    def _():
        pltpu.sync_copy(losses_hbm, vals_v)
        lane = lax.iota(jnp.int32, L)
        v = [vals_v[pl.ds(j * L, L)] for j in range(NV)]
        idx = [lane + j * L for j in range(NV)]
        big = jnp.int32(2 ** 30)
        outs = [jnp.zeros((L,), jnp.float32) for _ in range(TOPK_K // L)]
        for r in range(TOPK_K):
            t = v[0]
            for j in range(1, NV):
                t = jnp.maximum(t, v[j])
            m = jnp.max(t)
            c = jnp.where(v[0] == m, idx[0], big)
            for j in range(1, NV):
                c = jnp.minimum(c, jnp.where(v[j] == m, idx[j], big))
            mi = jnp.min(c)
            for j in range(NV):
                v[j] = jnp.where(idx[j] == mi, jnp.float32(-1.0), v[j])
            q, p = divmod(r, L)
            outs[q] = jnp.where(lane == p, m, outs[q])
        for q in range(TOPK_K // L):
            out_v[pl.ds(q * L, L)] = outs[q]
        pltpu.sync_copy(out_v, out_hbm)


@functools.partial(
    pl.kernel,
    out_type=jax.ShapeDtypeStruct((TOPK_K,), jnp.float32),
    mesh=plsc.VectorSubcoreMesh(core_axis_name="c", subcore_axis_name="s"),
    compiler_params=pltpu.CompilerParams(needs_layout_passes=False),
    scratch_types=[
        pltpu.VMEM((B,), jnp.float32),
        pltpu.VMEM((TOPK_K,), jnp.float32),
    ],
)
def _topk_sc(losses_hbm, out_hbm, vals_v, out_v):
    _topk_sc_body(losses_hbm, out_hbm, vals_v, out_v)


def kernel(output, label):
    return _topk_sc(_losses(output, label))
